# baseline (speedup 1.0000x reference)
import jax
import jax.numpy as jnp
from jax import lax
from jax.experimental import pallas as pl
from jax.experimental.pallas import tpu as pltpu
from jax.experimental.pallas import tpu_sc as plsc

B = 4096
EMB = 64
H1, H2 = 256, 128
NC, NS = 2, 16
NW = NC * NS
BPW = B // NW


def _sc_gather_body(idx_hbm, table_hbm, out_hbm, idx_v, rows_v, sem):
    wid = lax.axis_index("s") * NC + lax.axis_index("c")
    base = wid * BPW
    pltpu.sync_copy(idx_hbm.at[pl.ds(base, BPW)], idx_v)
    pltpu.async_copy(table_hbm.at[idx_v], rows_v, sem).wait()
    pltpu.sync_copy(rows_v, out_hbm.at[pl.ds(base, BPW)])


def _sc_gather(idx, table):
    mesh = plsc.VectorSubcoreMesh(core_axis_name="c", subcore_axis_name="s")
    return pl.kernel(
        _sc_gather_body,
        mesh=mesh,
        out_type=jax.ShapeDtypeStruct((B, EMB), jnp.float32),
        compiler_params=pltpu.CompilerParams(use_tc_tiling_on_sc=False),
        scratch_types=[
            pltpu.VMEM((BPW,), jnp.int32),
            pltpu.VMEM((BPW, EMB), jnp.float32),
            pltpu.SemaphoreType.DMA,
        ],
    )(idx, table)


def _small_body(sec, stg, reg, nums, sec_t, stg_t, reg_t, w1, b1, hs):
    f32 = jnp.float32
    sec_oh = (sec[...] == lax.broadcasted_iota(jnp.int32, (B, 32), 1)).astype(f32)
    stg_oh = (stg[...] == lax.broadcasted_iota(jnp.int32, (B, 16), 1)).astype(f32)
    reg_oh = (reg[...] == lax.broadcasted_iota(jnp.int32, (B, 24), 1)).astype(f32)

    w1v = w1[...]
    sec_w = jnp.dot(sec_t[...], w1v[64:80], preferred_element_type=f32)
    stg_w = jnp.dot(stg_t[...], w1v[80:96], preferred_element_type=f32)
    reg_w = jnp.dot(reg_t[...], w1v[96:112], preferred_element_type=f32)

    hs[...] = (jnp.dot(sec_oh, sec_w, preferred_element_type=f32)
               + jnp.dot(stg_oh, stg_w, preferred_element_type=f32)
               + jnp.dot(reg_oh, reg_w, preferred_element_type=f32)
               + jnp.dot(nums[...], w1v[112:118], preferred_element_type=f32)
               + b1[...])


def _tower_body(id_emb, hs, w1, g1, beta1, w2, b2, g2, beta2, out):
    f32 = jnp.float32
    h = jnp.dot(id_emb[...], w1[0:64], preferred_element_type=f32) + hs[...]
    h = jnp.maximum(h, 0.0)
    mu = jnp.mean(h, axis=0, keepdims=True)
    var = jnp.mean((h - mu) ** 2, axis=0, keepdims=True)
    h = g1[...] * (h - mu) * lax.rsqrt(var + 1e-5) + beta1[...]

    h = jnp.dot(h, w2[...], preferred_element_type=f32) + b2[...]
    h = jnp.maximum(h, 0.0)
    mu2 = jnp.mean(h, axis=0, keepdims=True)
    var2 = jnp.mean((h - mu2) ** 2, axis=0, keepdims=True)
    h = g2[...] * (h - mu2) * lax.rsqrt(var2 + 1e-5) + beta2[...]

    norm = jnp.sqrt(jnp.sum(h * h, axis=1, keepdims=True))
    out[...] = h / jnp.maximum(norm, 1e-12)


def kernel(id, sector, stage, region, deal_size, revenue_multiple,
           growth_rate, profitability, team_experience, market_size,
           deal_table, sector_table, stage_table, region_table,
           W1, b1, g1, beta1, W2, b2, g2, beta2):
    f32 = jnp.float32
    id_emb = _sc_gather(id.astype(jnp.int32), deal_table)
    nums = jnp.stack([deal_size, revenue_multiple, growth_rate, profitability,
                      team_experience, market_size], axis=-1)
    hs = pl.pallas_call(
        _small_body,
        out_shape=jax.ShapeDtypeStruct((B, H1), f32),
    )(sector.reshape(B, 1).astype(jnp.int32),
      stage.reshape(B, 1).astype(jnp.int32),
      region.reshape(B, 1).astype(jnp.int32),
      nums, sector_table, stage_table, region_table,
      W1, b1.reshape(1, H1))
    return pl.pallas_call(
        _tower_body,
        out_shape=jax.ShapeDtypeStruct((B, H2), f32),
    )(id_emb, hs, W1,
      g1.reshape(1, H1), beta1.reshape(1, H1),
      W2, b2.reshape(1, H2), g2.reshape(1, H2), beta2.reshape(1, H2))


# speedup vs baseline: 1.1289x; 1.1289x over previous
import jax
import jax.numpy as jnp
from jax import lax
from jax.experimental import pallas as pl
from jax.experimental.pallas import tpu as pltpu
from jax.experimental.pallas import tpu_sc as plsc

B = 4096
EMB = 64
H1, H2 = 256, 128
NC, NS = 2, 16
NW = NC * NS
BPW = B // NW


def _sc_gather_body(idx_hbm, table_hbm, out_hbm, idx_v, rows_v, sem):
    wid = lax.axis_index("s") * NC + lax.axis_index("c")
    base = wid * BPW
    pltpu.sync_copy(idx_hbm.at[pl.ds(base, BPW)], idx_v)
    pltpu.async_copy(table_hbm.at[idx_v], rows_v, sem).wait()
    pltpu.sync_copy(rows_v, out_hbm.at[pl.ds(base, BPW)])


def _sc_gather(idx, table):
    mesh = plsc.VectorSubcoreMesh(core_axis_name="c", subcore_axis_name="s")
    return pl.kernel(
        _sc_gather_body,
        mesh=mesh,
        out_type=jax.ShapeDtypeStruct((B, EMB), jnp.float32),
        compiler_params=pltpu.CompilerParams(use_tc_tiling_on_sc=False),
        scratch_types=[
            pltpu.VMEM((BPW,), jnp.int32),
            pltpu.VMEM((BPW, EMB), jnp.float32),
            pltpu.SemaphoreType.DMA,
        ],
    )(idx, table)


def _mlp_body(id_emb, sec, stg, reg, nums,
              sec_t, stg_t, reg_t,
              w1, b1, g1, beta1, w2, b2, g2, beta2, out):
    f32 = jnp.float32

    sec2 = sec[...].reshape(B, 1)
    stg2 = stg[...].reshape(B, 1)
    reg2 = reg[...].reshape(B, 1)

    sec_oh = (sec2 == lax.broadcasted_iota(jnp.int32, (B, 32), 1)).astype(f32)
    stg_oh = (stg2 == lax.broadcasted_iota(jnp.int32, (B, 16), 1)).astype(f32)
    reg_oh = (reg2 == lax.broadcasted_iota(jnp.int32, (B, 24), 1)).astype(f32)

    w1v = w1[...]
    sec_w = jnp.dot(sec_t[...], w1v[64:80], preferred_element_type=f32)
    stg_w = jnp.dot(stg_t[...], w1v[80:96], preferred_element_type=f32)
    reg_w = jnp.dot(reg_t[...], w1v[96:112], preferred_element_type=f32)

    h = (jnp.dot(id_emb[...], w1v[0:64], preferred_element_type=f32)
         + jnp.dot(sec_oh, sec_w, preferred_element_type=f32)
         + jnp.dot(stg_oh, stg_w, preferred_element_type=f32)
         + jnp.dot(reg_oh, reg_w, preferred_element_type=f32)
         + jnp.dot(nums[...], w1v[112:118], preferred_element_type=f32)
         + b1[...])
    h = jnp.maximum(h, 0.0)
    mu = jnp.mean(h, axis=0, keepdims=True)
    var = jnp.mean((h - mu) ** 2, axis=0, keepdims=True)
    h = g1[...] * (h - mu) * lax.rsqrt(var + 1e-5) + beta1[...]

    h = jnp.dot(h, w2[...], preferred_element_type=f32) + b2[...]
    h = jnp.maximum(h, 0.0)
    mu2 = jnp.mean(h, axis=0, keepdims=True)
    var2 = jnp.mean((h - mu2) ** 2, axis=0, keepdims=True)
    h = g2[...] * (h - mu2) * lax.rsqrt(var2 + 1e-5) + beta2[...]

    norm = jnp.sqrt(jnp.sum(h * h, axis=1, keepdims=True))
    out[...] = h / jnp.maximum(norm, 1e-12)


def kernel(id, sector, stage, region, deal_size, revenue_multiple,
           growth_rate, profitability, team_experience, market_size,
           deal_table, sector_table, stage_table, region_table,
           W1, b1, g1, beta1, W2, b2, g2, beta2):
    f32 = jnp.float32
    id_emb = _sc_gather(id.astype(jnp.int32), deal_table)
    nums = jnp.stack([deal_size, revenue_multiple, growth_rate, profitability,
                      team_experience, market_size], axis=-1)
    return pl.pallas_call(
        _mlp_body,
        out_shape=jax.ShapeDtypeStruct((B, H2), f32),
    )(id_emb, sector.astype(jnp.int32), stage.astype(jnp.int32),
      region.astype(jnp.int32), nums, sector_table, stage_table, region_table,
      W1, b1, g1, beta1, W2, b2, g2, beta2)


# speedup vs baseline: 1.1521x; 1.0205x over previous
import jax
import jax.numpy as jnp
from jax import lax
from jax.experimental import pallas as pl
from jax.experimental.pallas import tpu as pltpu
from jax.experimental.pallas import tpu_sc as plsc

B = 4096
EMB = 64
H1, H2 = 256, 128
NC, NS = 2, 16
NW = NC * NS
BPW = B // NW


def _sc_gather_body(idx_hbm, table2_hbm, out_hbm, idx_v, sup_v, rows_v, sem):
    wid = lax.axis_index("s") * NC + lax.axis_index("c")
    base = wid * BPW
    pltpu.sync_copy(idx_hbm.at[pl.ds(base, BPW)], idx_v)
    for j in range(BPW // 16):
        v = idx_v[pl.ds(j * 16, 16)]
        sup_v[pl.ds(j * 16, 16)] = lax.shift_right_logical(v, 1)
    pltpu.async_copy(table2_hbm.at[sup_v], rows_v, sem).wait()
    pltpu.sync_copy(rows_v, out_hbm.at[pl.ds(base, BPW)])


def _sc_gather(idx, table2):
    mesh = plsc.VectorSubcoreMesh(core_axis_name="c", subcore_axis_name="s")
    return pl.kernel(
        _sc_gather_body,
        mesh=mesh,
        out_type=jax.ShapeDtypeStruct((B, 2 * EMB), jnp.float32),
        compiler_params=pltpu.CompilerParams(use_tc_tiling_on_sc=False),
        scratch_types=[
            pltpu.VMEM((BPW,), jnp.int32),
            pltpu.VMEM((BPW,), jnp.int32),
            pltpu.VMEM((BPW, 2 * EMB), jnp.float32),
            pltpu.SemaphoreType.DMA,
        ],
    )(idx, table2)


def _mlp_body(x2, ids, sec, stg, reg, nums,
              sec_t, stg_t, reg_t,
              w1, b1, g1, beta1, w2, b2, g2, beta2, out):
    f32 = jnp.float32

    odd = lax.bitwise_and(ids[...], 1).reshape(B, 1)
    x2v = x2[...]
    id_emb = jnp.where(odd == 1, x2v[:, EMB:2 * EMB], x2v[:, 0:EMB])

    sec2 = sec[...].reshape(B, 1)
    stg2 = stg[...].reshape(B, 1)
    reg2 = reg[...].reshape(B, 1)

    sec_oh = (sec2 == lax.broadcasted_iota(jnp.int32, (B, 32), 1)).astype(f32)
    stg_oh = (stg2 == lax.broadcasted_iota(jnp.int32, (B, 16), 1)).astype(f32)
    reg_oh = (reg2 == lax.broadcasted_iota(jnp.int32, (B, 24), 1)).astype(f32)

    w1v = w1[...]
    sec_w = jnp.dot(sec_t[...], w1v[64:80], preferred_element_type=f32)
    stg_w = jnp.dot(stg_t[...], w1v[80:96], preferred_element_type=f32)
    reg_w = jnp.dot(reg_t[...], w1v[96:112], preferred_element_type=f32)

    h = (jnp.dot(id_emb, w1v[0:64], preferred_element_type=f32)
         + jnp.dot(sec_oh, sec_w, preferred_element_type=f32)
         + jnp.dot(stg_oh, stg_w, preferred_element_type=f32)
         + jnp.dot(reg_oh, reg_w, preferred_element_type=f32)
         + jnp.dot(nums[...], w1v[112:118], preferred_element_type=f32)
         + b1[...])
    h = jnp.maximum(h, 0.0)
    mu = jnp.mean(h, axis=0, keepdims=True)
    var = jnp.mean((h - mu) ** 2, axis=0, keepdims=True)
    h = g1[...] * (h - mu) * lax.rsqrt(var + 1e-5) + beta1[...]

    h = jnp.dot(h, w2[...], preferred_element_type=f32) + b2[...]
    h = jnp.maximum(h, 0.0)
    mu2 = jnp.mean(h, axis=0, keepdims=True)
    var2 = jnp.mean((h - mu2) ** 2, axis=0, keepdims=True)
    h = g2[...] * (h - mu2) * lax.rsqrt(var2 + 1e-5) + beta2[...]

    norm = jnp.sqrt(jnp.sum(h * h, axis=1, keepdims=True))
    out[...] = h / jnp.maximum(norm, 1e-12)


def kernel(id, sector, stage, region, deal_size, revenue_multiple,
           growth_rate, profitability, team_experience, market_size,
           deal_table, sector_table, stage_table, region_table,
           W1, b1, g1, beta1, W2, b2, g2, beta2):
    f32 = jnp.float32
    idx = id.astype(jnp.int32)
    x2 = _sc_gather(idx, deal_table.reshape(50000, 2 * EMB))
    nums = jnp.stack([deal_size, revenue_multiple, growth_rate, profitability,
                      team_experience, market_size], axis=-1)
    return pl.pallas_call(
        _mlp_body,
        out_shape=jax.ShapeDtypeStruct((B, H2), f32),
    )(x2, idx, sector.astype(jnp.int32), stage.astype(jnp.int32),
      region.astype(jnp.int32), nums, sector_table, stage_table, region_table,
      W1, b1, g1, beta1, W2, b2, g2, beta2)
